# Initial kernel scaffold; baseline (speedup 1.0000x reference)
import jax
import jax.numpy as jnp
from jax.experimental import pallas as pl


def kernel(x, table):
    raise NotImplementedError("write your pallas kernel here")


# speedup vs baseline: 4.7605x; 4.7605x over previous
import jax
import jax.numpy as jnp
from jax.experimental import pallas as pl
from jax.experimental.pallas import tpu as pltpu


def kernel(x, table):
    b, s, d = x.shape
    dim = table.shape[1]
    scale = float(dim ** 0.5)
    block_s = 256

    def body(t_ref, o_ref):
        scaled = t_ref[...] * scale
        o_ref[...] = jnp.broadcast_to(scaled[None], o_ref.shape)

    out = pl.pallas_call(
        body,
        grid=(s // block_s,),
        in_specs=[pl.BlockSpec((block_s, d), lambda i: (i, 0))],
        out_specs=pl.BlockSpec((b, block_s, d), lambda i: (0, i, 0)),
        out_shape=jax.ShapeDtypeStruct((b, s, d), table.dtype),
        compiler_params=pltpu.CompilerParams(
            dimension_semantics=("arbitrary",),
        ),
    )(table[:s])
    return out
